# baseline (speedup 1.0000x reference)
import functools

import jax
import jax.numpy as jnp
from jax import lax
from jax.experimental import pallas as pl
from jax.experimental.pallas import tpu as pltpu
from jax.experimental.pallas import tpu_sc as plsc

N = 10000
E = 160000
D = 512
H = 8
DH = 64
L = 2
EPS = 1e-5
NEG_SLOPE = 0.2

NC = 2
NS = 16
NW = NC * NS
EPAD = 163840
EPW = EPAD // NW
CA = 64
CB = 512
CC = 128
RPT = N // NS
RS = EPAD * H // 128

_sc_mesh = plsc.VectorSubcoreMesh(core_axis_name="c", subcore_axis_name="s")

_GDN = lax.GatherDimensionNumbers(
    offset_dims=(), collapsed_slice_dims=(0,), start_index_map=(0,))


def _lane_bcast(v16, i):
    idx = jnp.zeros((16,), jnp.int32) + i
    return lax.gather(v16, idx[:, None], _GDN, (1,),
                      mode=lax.GatherScatterMode.PROMISE_IN_BOUNDS)


def _proj_body(x_ref, ws_ref, wd_ref, fs_ref, fd_ref):
    x = x_ref[...]
    fs_ref[...] = jnp.dot(x, ws_ref[...], preferred_element_type=jnp.float32)
    fd_ref[...] = jnp.dot(x, wd_ref[...], preferred_element_type=jnp.float32)


def _project(h, ws, wd):
    bm = 400
    return pl.pallas_call(
        _proj_body,
        grid=(N // bm,),
        in_specs=[
            pl.BlockSpec((bm, D), lambda i: (i, 0)),
            pl.BlockSpec((D, D), lambda i: (0, 0)),
            pl.BlockSpec((D, D), lambda i: (0, 0)),
        ],
        out_specs=[
            pl.BlockSpec((bm, D), lambda i: (i, 0)),
            pl.BlockSpec((bm, D), lambda i: (i, 0)),
        ],
        out_shape=[jax.ShapeDtypeStruct((N, D), jnp.float32)] * 2,
    )(h, ws, wd)


@functools.partial(
    pl.kernel,
    out_type=jax.ShapeDtypeStruct((EPAD * H,), jnp.float32),
    mesh=_sc_mesh,
    scratch_types=[
        pltpu.VMEM((CA,), jnp.int32),
        pltpu.VMEM((CA,), jnp.int32),
        pltpu.VMEM((CA, D), jnp.float32),
        pltpu.VMEM((CA, D), jnp.float32),
        pltpu.VMEM((H * DH,), jnp.float32),
        pltpu.VMEM((CA * H,), jnp.float32),
        pltpu.SemaphoreType.DMA,
    ],
    compiler_params=pltpu.CompilerParams(use_tc_tiling_on_sc=False, needs_layout_passes=False),
)
def _score_kernel(fs_hbm, fd_hbm, src_hbm, dst_hbm, attn_hbm, scores_hbm,
                  sidx_v, didx_v, fs_v, fd_v, attn_v, sc_v, sem):
    cid = lax.axis_index("c")
    sid = lax.axis_index("s")
    wid = cid * NS + sid
    ebase = wid * EPW
    pltpu.sync_copy(attn_hbm, attn_v)
    ilane = lax.iota(jnp.int32, 16)

    def chunk_body(c, _):
        eb = ebase + c * CA
        pltpu.sync_copy(src_hbm.at[pl.ds(eb, CA)], sidx_v)
        pltpu.sync_copy(dst_hbm.at[pl.ds(eb, CA)], didx_v)
        pltpu.async_copy(fs_hbm.at[sidx_v], fs_v, sem).wait()
        pltpu.async_copy(fd_hbm.at[didx_v], fd_v, sem).wait()

        def group_body(g, _):
            eids = ilane + g * 16
            eids_h = eids * H

            def head_body(h, _):
                hbase = h * DH
                attn16s = [attn_v[pl.ds(hbase + q * 16, 16)]
                           for q in range(4)]
                colbs = [jnp.zeros((16,), jnp.int32) + (hbase + q * 16)
                         for q in range(4)]
                accs = [jnp.zeros((16,), jnp.float32) for _ in range(4)]
                for k2 in range(16):
                    for q in range(4):
                        colv = colbs[q] + k2
                        a = plsc.load_gather(fs_v, [eids, colv])
                        b = plsc.load_gather(fd_v, [eids, colv])
                        z = a + b
                        z = jnp.maximum(z, NEG_SLOPE * z)
                        s16 = _lane_bcast(attn16s[q], k2)
                        accs[q] = accs[q] + s16 * z
                acc = (accs[0] + accs[1]) + (accs[2] + accs[3])
                plsc.store_scatter(sc_v, [eids_h + h], acc)
                return 0

            lax.fori_loop(0, H, head_body, 0)
            return 0

        lax.fori_loop(0, CA // 16, group_body, 0)
        pltpu.sync_copy(sc_v, scores_hbm.at[pl.ds(eb * H, CA * H)])
        return 0

    lax.fori_loop(0, EPW // CA, chunk_body, 0)


def _softmax_body(s_ref, ex_ref):
    s = s_ref[...]
    m = jnp.max(s, axis=0, keepdims=True)
    mh = m[:, 0:8]
    for i in range(1, 16):
        mh = jnp.maximum(mh, m[:, 8 * i:8 * i + 8])
    mg = jnp.concatenate([mh] * 16, axis=1)
    ex = jnp.exp(s - mg)
    row = lax.broadcasted_iota(jnp.int32, (RS, 128), 0)
    ex_ref[...] = jnp.where(row < E // 16, ex, 0.0)


def _softmax_prep(scores):
    return pl.pallas_call(
        _softmax_body,
        out_shape=jax.ShapeDtypeStruct((RS, 128), jnp.float32),
    )(scores.reshape(RS, 128))


@functools.partial(
    pl.kernel,
    out_type=jax.ShapeDtypeStruct((NC * N, H), jnp.float32),
    mesh=_sc_mesh,
    scratch_types=[
        pltpu.VMEM((CB,), jnp.int32),
        pltpu.VMEM((CB, H), jnp.float32),
        pltpu.VMEM_SHARED((N, H), jnp.float32),
    ],
    compiler_params=pltpu.CompilerParams(use_tc_tiling_on_sc=False, needs_layout_passes=False),
)
def _denom_kernel(ex_hbm, dst_hbm, zeros_hbm, denomp_hbm,
                  didx_v, ex_v, den_s):
    cid = lax.axis_index("c")
    sid = lax.axis_index("s")
    wid = cid * NS + sid
    pltpu.sync_copy(zeros_hbm.at[pl.ds(sid * RPT, RPT), :],
                    den_s.at[pl.ds(sid * RPT, RPT), :])
    plsc.subcore_barrier()

    def chunk_body(c, _):
        eb = wid * EPW + c * CB
        pltpu.sync_copy(dst_hbm.at[pl.ds(eb, CB)], didx_v)
        pltpu.sync_copy(ex_hbm.at[pl.ds(eb, CB), :], ex_v)
        pltpu.sync_copy(ex_v, den_s.at[didx_v], add=True)
        return 0

    lax.fori_loop(0, EPW // CB, chunk_body, 0)
    plsc.subcore_barrier()
    pltpu.sync_copy(den_s.at[pl.ds(sid * RPT, RPT), :],
                    denomp_hbm.at[pl.ds(cid * N + sid * RPT, RPT), :])


@functools.partial(
    pl.kernel,
    out_type=jax.ShapeDtypeStruct((NC * N, D), jnp.float32),
    mesh=_sc_mesh,
    scratch_types=[
        pltpu.VMEM((CC,), jnp.int32),
        pltpu.VMEM((CC,), jnp.int32),
        pltpu.VMEM((CC,), jnp.int32),
        pltpu.VMEM((CC, H), jnp.float32),
        pltpu.VMEM((CC, DH), jnp.float32),
        pltpu.VMEM_SHARED((N, DH), jnp.float32),
        pltpu.SemaphoreType.DMA,
    ],
    compiler_params=pltpu.CompilerParams(use_tc_tiling_on_sc=False, needs_layout_passes=False),
)
def _message_kernel(fsh_hbm, src_hbm, dst_hbm, ex_hbm, zeros_hbm, outp_hbm,
                    sidx_v, didx_v, gidx_v, ex_v, rows_v, out_s, sem):
    cid = lax.axis_index("c")
    sid = lax.axis_index("s")
    wid = cid * NS + sid
    for h in range(H):
        pltpu.sync_copy(zeros_hbm.at[pl.ds(sid * RPT, RPT), :],
                        out_s.at[pl.ds(sid * RPT, RPT), :])
        plsc.subcore_barrier()

        def chunk_body(c, _, _h=h):
            eb = wid * EPW + c * CC
            pltpu.sync_copy(src_hbm.at[pl.ds(eb, CC)], sidx_v)
            pltpu.sync_copy(dst_hbm.at[pl.ds(eb, CC)], didx_v)
            pltpu.sync_copy(ex_hbm.at[pl.ds(eb, CC), :], ex_v)

            for j in range(CC // 16):
                v = sidx_v[pl.ds(j * 16, 16)]
                gidx_v[pl.ds(j * 16, 16)] = v * H + _h

            pltpu.async_copy(fsh_hbm.at[gidx_v], rows_v, sem).wait()
            ilane = lax.iota(jnp.int32, 16)
            hsplat = jnp.zeros((16,), jnp.int32) + _h

            def scale_body(j, _):
                eids = ilane + j * 16
                ex16 = plsc.load_gather(ex_v, [eids, hsplat])
                for col in range(DH):
                    colv = jnp.zeros((16,), jnp.int32) + col
                    r = plsc.load_gather(rows_v, [eids, colv])
                    plsc.store_scatter(rows_v, [eids, colv], r * ex16)
                return 0

            lax.fori_loop(0, CC // 16, scale_body, 0)
            pltpu.sync_copy(rows_v, out_s.at[didx_v], add=True)
            return 0

        lax.fori_loop(0, EPW // CC, chunk_body, 0)
        plsc.subcore_barrier()
        pltpu.sync_copy(out_s.at[pl.ds(sid * RPT, RPT), :],
                        outp_hbm.at[pl.ds(cid * N + sid * RPT, RPT),
                                    pl.ds(h * DH, DH)])


def _epi_body(final, o0_ref, o1_ref, d0_ref, d1_ref, hp_ref, bias_ref,
              g1_ref, b1_ref, mex_ref, g2_ref, b2_ref, out_ref):
    den = d0_ref[...] + d1_ref[...] + 1e-9
    dx = jnp.dot(den, mex_ref[...], preferred_element_type=jnp.float32)
    o = (o0_ref[...] + o1_ref[...]) / dx + bias_ref[...]
    o = jnp.where(o > 0, o, jnp.exp(o) - 1.0)
    t = o + hp_ref[...]
    mu = jnp.mean(t, axis=1, keepdims=True)
    var = jnp.mean((t - mu) * (t - mu), axis=1, keepdims=True)
    t = (t - mu) * lax.rsqrt(var + EPS) * g1_ref[...] + b1_ref[...]
    if final:
        mu2 = jnp.mean(t, axis=1, keepdims=True)
        var2 = jnp.mean((t - mu2) * (t - mu2), axis=1, keepdims=True)
        t = (t - mu2) * lax.rsqrt(var2 + EPS) * g2_ref[...] + b2_ref[...]
    out_ref[...] = t


def _epilogue(o0, o1, d0, d1, hprev, bias, g1, b1, mex, g2, b2, final):
    bm = 400
    row = pl.BlockSpec((bm, D), lambda i: (i, 0))
    vec = pl.BlockSpec((1, D), lambda i: (0, 0))
    return pl.pallas_call(
        functools.partial(_epi_body, final),
        grid=(N // bm,),
        in_specs=[
            row, row,
            pl.BlockSpec((bm, H), lambda i: (i, 0)),
            pl.BlockSpec((bm, H), lambda i: (i, 0)),
            row, vec, vec, vec,
            pl.BlockSpec((H, D), lambda i: (0, 0)),
            vec, vec,
        ],
        out_specs=row,
        out_shape=jax.ShapeDtypeStruct((N, D), jnp.float32),
    )(o0, o1, d0, d1, hprev, bias, g1, b1, mex, g2, b2)


def kernel(x, edge_index, W_src, W_dst, attn, gat_bias, ln_g, ln_b,
           outn_g, outn_b):
    src = edge_index[0].astype(jnp.int32)
    dst = edge_index[1].astype(jnp.int32)
    pad = jnp.zeros((EPAD - E,), jnp.int32)
    srcp = jnp.concatenate([src, pad])
    dstp = jnp.concatenate([dst, pad])
    zeros_nh = jnp.zeros((N, H), jnp.float32)
    zeros_nd = jnp.zeros((N, DH), jnp.float32)
    mex = jnp.repeat(jnp.eye(H, dtype=jnp.float32), DH, axis=1)

    h = x
    for l in range(L):
        fs, fd = _project(h, W_src[l], W_dst[l])
        scores = _score_kernel(fs, fd, srcp, dstp, attn[l].reshape(-1))
        ex = _softmax_prep(scores).reshape(EPAD, H)
        denomp = _denom_kernel(ex, dstp, zeros_nh)
        outp = _message_kernel(fs.reshape(N * H, DH), srcp, dstp, ex,
                               zeros_nd)
        h = _epilogue(outp[:N], outp[N:], denomp[:N], denomp[N:], h,
                      gat_bias[l].reshape(1, D), ln_g[l].reshape(1, D),
                      ln_b[l].reshape(1, D), mex,
                      outn_g.reshape(1, D), outn_b.reshape(1, D),
                      final=(l == L - 1))
    return h


# speedup vs baseline: 2.3971x; 2.3971x over previous
import functools

import jax
import jax.numpy as jnp
from jax import lax
from jax.experimental import pallas as pl
from jax.experimental.pallas import tpu as pltpu
from jax.experimental.pallas import tpu_sc as plsc

N = 10000
E = 160000
D = 512
H = 8
DH = 64
L = 2
EPS = 1e-5
NEG_SLOPE = 0.2

NC = 2
NS = 16
NW = NC * NS
EPAD = 163840
EPW = EPAD // NW
CA = 64
CB = 512
CC = 128
RPT = N // NS
RS = EPAD * H // 128

_sc_mesh = plsc.VectorSubcoreMesh(core_axis_name="c", subcore_axis_name="s")

_GDN = lax.GatherDimensionNumbers(
    offset_dims=(), collapsed_slice_dims=(0,), start_index_map=(0,))


def _lane_bcast(v16, i):
    idx = jnp.zeros((16,), jnp.int32) + i
    return lax.gather(v16, idx[:, None], _GDN, (1,),
                      mode=lax.GatherScatterMode.PROMISE_IN_BOUNDS)


def _lane_perm(v16, idx16):
    return lax.gather(v16, idx16[:, None], _GDN, (1,),
                      mode=lax.GatherScatterMode.PROMISE_IN_BOUNDS)


def _lane_sum(v16, perms):
    for pm in perms:
        v16 = v16 + _lane_perm(v16, pm)
    return v16


def _proj_body(x_ref, ws_ref, wd_ref, fs_ref, fd_ref):
    x = x_ref[...]
    fs_ref[...] = jnp.dot(x, ws_ref[...], preferred_element_type=jnp.float32)
    fd_ref[...] = jnp.dot(x, wd_ref[...], preferred_element_type=jnp.float32)


def _project(h, ws, wd):
    bm = 400
    return pl.pallas_call(
        _proj_body,
        grid=(N // bm,),
        in_specs=[
            pl.BlockSpec((bm, D), lambda i: (i, 0)),
            pl.BlockSpec((D, D), lambda i: (0, 0)),
            pl.BlockSpec((D, D), lambda i: (0, 0)),
        ],
        out_specs=[
            pl.BlockSpec((bm, D), lambda i: (i, 0)),
            pl.BlockSpec((bm, D), lambda i: (i, 0)),
        ],
        out_shape=[jax.ShapeDtypeStruct((N, D), jnp.float32)] * 2,
    )(h, ws, wd)


@functools.partial(
    pl.kernel,
    out_type=jax.ShapeDtypeStruct((H, EPAD), jnp.float32),
    mesh=_sc_mesh,
    scratch_types=[
        pltpu.VMEM((CA,), jnp.int32),
        pltpu.VMEM((CA,), jnp.int32),
        pltpu.VMEM((CA, D), jnp.float32),
        pltpu.VMEM((CA, D), jnp.float32),
        pltpu.VMEM((H * DH,), jnp.float32),
        pltpu.VMEM((H, CA), jnp.float32),
        pltpu.SemaphoreType.DMA,
    ],
    compiler_params=pltpu.CompilerParams(use_tc_tiling_on_sc=False, needs_layout_passes=False),
)
def _score_kernel(fs_hbm, fd_hbm, src_hbm, dst_hbm, attn_hbm, scores_hbm,
                  sidx_v, didx_v, fs_v, fd_v, attn_v, sc_v, sem):
    cid = lax.axis_index("c")
    sid = lax.axis_index("s")
    wid = cid * NS + sid
    ebase = wid * EPW
    pltpu.sync_copy(attn_hbm, attn_v)
    ilane = lax.iota(jnp.int32, 16)
    perms = [ilane ^ m for m in (1, 2, 4, 8)]

    def chunk_body(c, _):
        eb = ebase + c * CA
        pltpu.sync_copy(src_hbm.at[pl.ds(eb, CA)], sidx_v)
        pltpu.sync_copy(dst_hbm.at[pl.ds(eb, CA)], didx_v)
        pltpu.async_copy(fs_hbm.at[sidx_v], fs_v, sem).wait()
        pltpu.async_copy(fd_hbm.at[didx_v], fd_v, sem).wait()

        def head_body(h, _):
            hbase = h * DH
            at = [attn_v[pl.ds(hbase + q * 16, 16)] for q in range(4)]

            def group_body(g, _):
                scores16 = jnp.zeros((16,), jnp.float32)
                for e2 in range(16):
                    e = g * 16 + e2
                    ps = []
                    for q in range(4):
                        a = fs_v[e, pl.ds(hbase + q * 16, 16)]
                        b = fd_v[e, pl.ds(hbase + q * 16, 16)]
                        z = a + b
                        z = jnp.maximum(z, NEG_SLOPE * z)
                        ps.append(at[q] * z)
                    p = (ps[0] + ps[1]) + (ps[2] + ps[3])
                    p = _lane_sum(p, perms)
                    scores16 = jnp.where(ilane == e2, p, scores16)
                sc_v[h, pl.ds(g * 16, 16)] = scores16
                return 0

            lax.fori_loop(0, CA // 16, group_body, 0)
            return 0

        lax.fori_loop(0, H, head_body, 0)
        pltpu.sync_copy(sc_v, scores_hbm.at[:, pl.ds(eb, CA)])
        return 0

    lax.fori_loop(0, EPW // CA, chunk_body, 0)


def _softmax_body(s_ref, ex_ref):
    HB = EPAD // 128
    EB = E // 128
    riot = lax.broadcasted_iota(jnp.int32, (HB, 128), 0)
    for h in range(H):
        s = s_ref[pl.ds(h * HB, HB), :]
        mh = jnp.max(s)
        exs = jnp.exp(s - mh)
        ex_ref[pl.ds(h * HB, HB), :] = jnp.where(riot < EB, exs, 0.0)


def _softmax_prep(scores):
    return pl.pallas_call(
        _softmax_body,
        out_shape=jax.ShapeDtypeStruct((RS, 128), jnp.float32),
    )(scores.reshape(RS, 128))


def _tr_body(x_ref, o_ref):
    o_ref[...] = x_ref[...].T


def _transpose_ex(ex_hm):
    bt = 8192
    return pl.pallas_call(
        _tr_body,
        grid=(EPAD // bt,),
        in_specs=[pl.BlockSpec((H, bt), lambda i: (0, i))],
        out_specs=pl.BlockSpec((bt, H), lambda i: (i, 0)),
        out_shape=jax.ShapeDtypeStruct((EPAD, H), jnp.float32),
    )(ex_hm)


@functools.partial(
    pl.kernel,
    out_type=jax.ShapeDtypeStruct((NC * N, H), jnp.float32),
    mesh=_sc_mesh,
    scratch_types=[
        pltpu.VMEM((CB,), jnp.int32),
        pltpu.VMEM((CB, H), jnp.float32),
        pltpu.VMEM_SHARED((N, H), jnp.float32),
    ],
    compiler_params=pltpu.CompilerParams(use_tc_tiling_on_sc=False, needs_layout_passes=False),
)
def _denom_kernel(ex_hbm, dst_hbm, zeros_hbm, denomp_hbm,
                  didx_v, ex_v, den_s):
    cid = lax.axis_index("c")
    sid = lax.axis_index("s")
    wid = cid * NS + sid
    pltpu.sync_copy(zeros_hbm.at[pl.ds(sid * RPT, RPT), :],
                    den_s.at[pl.ds(sid * RPT, RPT), :])
    plsc.subcore_barrier()

    def chunk_body(c, _):
        eb = wid * EPW + c * CB
        pltpu.sync_copy(dst_hbm.at[pl.ds(eb, CB)], didx_v)
        pltpu.sync_copy(ex_hbm.at[pl.ds(eb, CB), :], ex_v)
        pltpu.sync_copy(ex_v, den_s.at[didx_v], add=True)
        return 0

    lax.fori_loop(0, EPW // CB, chunk_body, 0)
    plsc.subcore_barrier()
    pltpu.sync_copy(den_s.at[pl.ds(sid * RPT, RPT), :],
                    denomp_hbm.at[pl.ds(cid * N + sid * RPT, RPT), :])


@functools.partial(
    pl.kernel,
    out_type=jax.ShapeDtypeStruct((NC * N, D), jnp.float32),
    mesh=_sc_mesh,
    scratch_types=[
        pltpu.VMEM((CC,), jnp.int32),
        pltpu.VMEM((CC,), jnp.int32),
        pltpu.VMEM((CC,), jnp.int32),
        pltpu.VMEM((CC,), jnp.float32),
        pltpu.VMEM((CC, DH), jnp.float32),
        pltpu.VMEM_SHARED((N, DH), jnp.float32),
        pltpu.SemaphoreType.DMA,
    ],
    compiler_params=pltpu.CompilerParams(use_tc_tiling_on_sc=False, needs_layout_passes=False),
)
def _message_kernel(fsh_hbm, src_hbm, dst_hbm, exh_hbm, zeros_hbm, outp_hbm,
                    sidx_v, didx_v, gidx_v, exh_v, rows_v, out_s, sem):
    cid = lax.axis_index("c")
    sid = lax.axis_index("s")
    wid = cid * NS + sid
    for h in range(H):
        pltpu.sync_copy(zeros_hbm.at[pl.ds(sid * RPT, RPT), :],
                        out_s.at[pl.ds(sid * RPT, RPT), :])
        plsc.subcore_barrier()

        def chunk_body(c, _, _h=h):
            eb = wid * EPW + c * CC
            pltpu.sync_copy(src_hbm.at[pl.ds(eb, CC)], sidx_v)
            pltpu.sync_copy(dst_hbm.at[pl.ds(eb, CC)], didx_v)
            pltpu.sync_copy(exh_hbm.at[pl.ds(_h * EPAD + eb, CC)], exh_v)

            for j in range(CC // 16):
                v = sidx_v[pl.ds(j * 16, 16)]
                gidx_v[pl.ds(j * 16, 16)] = v * H + _h

            pltpu.async_copy(fsh_hbm.at[gidx_v], rows_v, sem).wait()

            def scale_body(j, _):
                ex16 = exh_v[pl.ds(j * 16, 16)]
                for e2 in range(16):
                    a16 = _lane_bcast(ex16, e2)
                    e = j * 16 + e2
                    for q in range(DH // 16):
                        rows_v[e, pl.ds(q * 16, 16)] = (
                            rows_v[e, pl.ds(q * 16, 16)] * a16)
                return 0

            lax.fori_loop(0, CC // 16, scale_body, 0)
            pltpu.sync_copy(rows_v, out_s.at[didx_v], add=True)
            return 0

        lax.fori_loop(0, EPW // CC, chunk_body, 0)
        plsc.subcore_barrier()
        pltpu.sync_copy(out_s.at[pl.ds(sid * RPT, RPT), :],
                        outp_hbm.at[pl.ds(cid * N + sid * RPT, RPT),
                                    pl.ds(h * DH, DH)])


def _epi_body(final, o0_ref, o1_ref, d0_ref, d1_ref, hp_ref, bias_ref,
              g1_ref, b1_ref, mex_ref, g2_ref, b2_ref, out_ref):
    den = d0_ref[...] + d1_ref[...] + 1e-9
    dx = jnp.dot(den, mex_ref[...], preferred_element_type=jnp.float32)
    o = (o0_ref[...] + o1_ref[...]) / dx + bias_ref[...]
    o = jnp.where(o > 0, o, jnp.exp(o) - 1.0)
    t = o + hp_ref[...]
    mu = jnp.mean(t, axis=1, keepdims=True)
    var = jnp.mean((t - mu) * (t - mu), axis=1, keepdims=True)
    t = (t - mu) * lax.rsqrt(var + EPS) * g1_ref[...] + b1_ref[...]
    if final:
        mu2 = jnp.mean(t, axis=1, keepdims=True)
        var2 = jnp.mean((t - mu2) * (t - mu2), axis=1, keepdims=True)
        t = (t - mu2) * lax.rsqrt(var2 + EPS) * g2_ref[...] + b2_ref[...]
    out_ref[...] = t


def _epilogue(o0, o1, d0, d1, hprev, bias, g1, b1, mex, g2, b2, final):
    bm = 400
    row = pl.BlockSpec((bm, D), lambda i: (i, 0))
    vec = pl.BlockSpec((1, D), lambda i: (0, 0))
    return pl.pallas_call(
        functools.partial(_epi_body, final),
        grid=(N // bm,),
        in_specs=[
            row, row,
            pl.BlockSpec((bm, H), lambda i: (i, 0)),
            pl.BlockSpec((bm, H), lambda i: (i, 0)),
            row, vec, vec, vec,
            pl.BlockSpec((H, D), lambda i: (0, 0)),
            vec, vec,
        ],
        out_specs=row,
        out_shape=jax.ShapeDtypeStruct((N, D), jnp.float32),
    )(o0, o1, d0, d1, hprev, bias, g1, b1, mex, g2, b2)


def kernel(x, edge_index, W_src, W_dst, attn, gat_bias, ln_g, ln_b,
           outn_g, outn_b):
    src = edge_index[0].astype(jnp.int32)
    dst = edge_index[1].astype(jnp.int32)
    pad = jnp.zeros((EPAD - E,), jnp.int32)
    srcp = jnp.concatenate([src, pad])
    dstp = jnp.concatenate([dst, pad])
    zeros_nh = jnp.zeros((N, H), jnp.float32)
    zeros_nd = jnp.zeros((N, DH), jnp.float32)
    mex = jnp.repeat(jnp.eye(H, dtype=jnp.float32), DH, axis=1)

    h = x
    for l in range(L):
        fs, fd = _project(h, W_src[l], W_dst[l])
        scores = _score_kernel(fs, fd, srcp, dstp, attn[l].reshape(-1))
        exf = _softmax_prep(scores.reshape(-1)).reshape(-1)
        ex_em = _transpose_ex(exf.reshape(H, EPAD))
        denomp = _denom_kernel(ex_em, dstp, zeros_nh)
        outp = _message_kernel(fs.reshape(N * H, DH), srcp, dstp, exf,
                               zeros_nd)
        h = _epilogue(outp[:N], outp[N:], denomp[:N], denomp[N:], h,
                      gat_bias[l].reshape(1, D), ln_g[l].reshape(1, D),
                      ln_b[l].reshape(1, D), mex,
                      outn_g.reshape(1, D), outn_b.reshape(1, D),
                      final=(l == L - 1))
    return h


# speedup vs baseline: 3.1112x; 1.2979x over previous
import functools

import jax
import jax.numpy as jnp
from jax import lax
from jax.experimental import pallas as pl
from jax.experimental.pallas import tpu as pltpu
from jax.experimental.pallas import tpu_sc as plsc

N = 10000
E = 160000
D = 512
H = 8
DH = 64
L = 2
EPS = 1e-5
NEG_SLOPE = 0.2

NC = 2
NS = 16
NW = NC * NS
EPAD = 163840
EPW = EPAD // NW
CA = 64
CB = 512
CC = 128
RPT = N // NS
RS = EPAD * H // 128

_sc_mesh = plsc.VectorSubcoreMesh(core_axis_name="c", subcore_axis_name="s")

_GDN = lax.GatherDimensionNumbers(
    offset_dims=(), collapsed_slice_dims=(0,), start_index_map=(0,))


def _lane_bcast(v16, i):
    idx = jnp.zeros((16,), jnp.int32) + i
    return lax.gather(v16, idx[:, None], _GDN, (1,),
                      mode=lax.GatherScatterMode.PROMISE_IN_BOUNDS)


def _lane_perm(v16, idx16):
    return lax.gather(v16, idx16[:, None], _GDN, (1,),
                      mode=lax.GatherScatterMode.PROMISE_IN_BOUNDS)


def _lane_sum(v16, perms):
    for pm in perms:
        v16 = v16 + _lane_perm(v16, pm)
    return v16


def _proj_body(x_ref, ws_ref, wd_ref, fs_ref, fd_ref):
    x = x_ref[...]
    fs_ref[...] = jnp.dot(x, ws_ref[...], preferred_element_type=jnp.float32)
    fd_ref[...] = jnp.dot(x, wd_ref[...], preferred_element_type=jnp.float32)


def _project(h, ws, wd):
    bm = 400
    return pl.pallas_call(
        _proj_body,
        grid=(N // bm,),
        in_specs=[
            pl.BlockSpec((bm, D), lambda i: (i, 0)),
            pl.BlockSpec((D, D), lambda i: (0, 0)),
            pl.BlockSpec((D, D), lambda i: (0, 0)),
        ],
        out_specs=[
            pl.BlockSpec((bm, D), lambda i: (i, 0)),
            pl.BlockSpec((bm, D), lambda i: (i, 0)),
        ],
        out_shape=[jax.ShapeDtypeStruct((N, D), jnp.float32)] * 2,
    )(h, ws, wd)


@functools.partial(
    pl.kernel,
    out_type=jax.ShapeDtypeStruct((H, EPAD), jnp.float32),
    mesh=_sc_mesh,
    scratch_types=[
        pltpu.VMEM((CA,), jnp.int32),
        pltpu.VMEM((CA,), jnp.int32),
        pltpu.VMEM((CA, D), jnp.float32),
        pltpu.VMEM((CA, D), jnp.float32),
        pltpu.VMEM((H * DH,), jnp.float32),
        pltpu.VMEM((H, CA), jnp.float32),
        pltpu.SemaphoreType.DMA,
    ],
    compiler_params=pltpu.CompilerParams(use_tc_tiling_on_sc=False, needs_layout_passes=False),
)
def _score_kernel(fs_hbm, fd_hbm, src_hbm, dst_hbm, attn_hbm, scores_hbm,
                  sidx_v, didx_v, fs_v, fd_v, attn_v, sc_v, sem):
    cid = lax.axis_index("c")
    sid = lax.axis_index("s")
    wid = cid * NS + sid
    ebase = wid * EPW
    pltpu.sync_copy(attn_hbm, attn_v)
    ilane = lax.iota(jnp.int32, 16)
    perms = [ilane ^ m for m in (1, 2, 4, 8)]

    def chunk_body(c, _):
        eb = ebase + c * CA
        pltpu.sync_copy(src_hbm.at[pl.ds(eb, CA)], sidx_v)
        pltpu.sync_copy(dst_hbm.at[pl.ds(eb, CA)], didx_v)
        pltpu.async_copy(fs_hbm.at[sidx_v], fs_v, sem).wait()
        pltpu.async_copy(fd_hbm.at[didx_v], fd_v, sem).wait()

        def head_body(h, _):
            hbase = h * DH
            at = [attn_v[pl.ds(hbase + q * 16, 16)] for q in range(4)]

            def group_body(g, _):
                scores16 = jnp.zeros((16,), jnp.float32)
                for e2 in range(16):
                    e = g * 16 + e2
                    ps = []
                    for q in range(4):
                        a = fs_v[e, pl.ds(hbase + q * 16, 16)]
                        b = fd_v[e, pl.ds(hbase + q * 16, 16)]
                        z = a + b
                        z = jnp.maximum(z, NEG_SLOPE * z)
                        ps.append(at[q] * z)
                    p = (ps[0] + ps[1]) + (ps[2] + ps[3])
                    p = _lane_sum(p, perms)
                    scores16 = jnp.where(ilane == e2, p, scores16)
                sc_v[h, pl.ds(g * 16, 16)] = scores16
                return 0

            lax.fori_loop(0, CA // 16, group_body, 0)
            return 0

        lax.fori_loop(0, H, head_body, 0)
        pltpu.sync_copy(sc_v, scores_hbm.at[:, pl.ds(eb, CA)])
        return 0

    lax.fori_loop(0, EPW // CA, chunk_body, 0)


def _softmax_body(s_ref, ex_ref):
    HB = EPAD // 128
    EB = E // 128
    riot = lax.broadcasted_iota(jnp.int32, (HB, 128), 0)
    for h in range(H):
        s = s_ref[pl.ds(h * HB, HB), :]
        mh = jnp.max(s)
        exs = jnp.exp(s - mh)
        ex_ref[pl.ds(h * HB, HB), :] = jnp.where(riot < EB, exs, 0.0)


def _softmax_prep(scores):
    return pl.pallas_call(
        _softmax_body,
        out_shape=jax.ShapeDtypeStruct((RS, 128), jnp.float32),
    )(scores.reshape(RS, 128))


def _tr_body(x_ref, o_ref):
    o_ref[...] = x_ref[...].T


def _transpose_ex(ex_hm):
    bt = 8192
    return pl.pallas_call(
        _tr_body,
        grid=(EPAD // bt,),
        in_specs=[pl.BlockSpec((H, bt), lambda i: (0, i))],
        out_specs=pl.BlockSpec((bt, H), lambda i: (i, 0)),
        out_shape=jax.ShapeDtypeStruct((EPAD, H), jnp.float32),
    )(ex_hm)


@functools.partial(
    pl.kernel,
    out_type=jax.ShapeDtypeStruct((NC * N, H), jnp.float32),
    mesh=_sc_mesh,
    scratch_types=[
        pltpu.VMEM((CB,), jnp.int32),
        pltpu.VMEM((CB, H), jnp.float32),
        pltpu.VMEM_SHARED((N, H), jnp.float32),
    ],
    compiler_params=pltpu.CompilerParams(use_tc_tiling_on_sc=False, needs_layout_passes=False),
)
def _denom_kernel(ex_hbm, dst_hbm, zeros_hbm, denomp_hbm,
                  didx_v, ex_v, den_s):
    cid = lax.axis_index("c")
    sid = lax.axis_index("s")
    wid = cid * NS + sid
    pltpu.sync_copy(zeros_hbm.at[pl.ds(sid * RPT, RPT), :],
                    den_s.at[pl.ds(sid * RPT, RPT), :])
    plsc.subcore_barrier()

    def chunk_body(c, _):
        eb = wid * EPW + c * CB
        pltpu.sync_copy(dst_hbm.at[pl.ds(eb, CB)], didx_v)
        pltpu.sync_copy(ex_hbm.at[pl.ds(eb, CB), :], ex_v)
        pltpu.sync_copy(ex_v, den_s.at[didx_v], add=True)
        return 0

    lax.fori_loop(0, EPW // CB, chunk_body, 0)
    plsc.subcore_barrier()
    pltpu.sync_copy(den_s.at[pl.ds(sid * RPT, RPT), :],
                    denomp_hbm.at[pl.ds(cid * N + sid * RPT, RPT), :])


@functools.partial(
    pl.kernel,
    out_type=jax.ShapeDtypeStruct((NC * N, D), jnp.float32),
    mesh=_sc_mesh,
    scratch_types=[
        pltpu.VMEM((CC,), jnp.int32),
        pltpu.VMEM((CC,), jnp.int32),
        pltpu.VMEM((CC,), jnp.int32),
        pltpu.VMEM((2, CC), jnp.float32),
        pltpu.VMEM((CC, 2 * DH), jnp.float32),
        pltpu.VMEM_SHARED((N, 2 * DH), jnp.float32),
        pltpu.SemaphoreType.DMA,
    ],
    compiler_params=pltpu.CompilerParams(use_tc_tiling_on_sc=False, needs_layout_passes=False),
)
def _message_kernel(fsh_hbm, src_hbm, dst_hbm, exh_hbm, zeros_hbm, outp_hbm,
                    sidx_v, didx_v, gidx_v, exh_v, rows_v, out_s, sem):
    cid = lax.axis_index("c")
    sid = lax.axis_index("s")
    wid = cid * NS + sid
    for hp in range(H // 2):
        pltpu.sync_copy(zeros_hbm.at[pl.ds(sid * RPT, RPT), :],
                        out_s.at[pl.ds(sid * RPT, RPT), :])
        plsc.subcore_barrier()

        def chunk_body(c, _, _hp=hp):
            eb = wid * EPW + c * CC
            pltpu.sync_copy(src_hbm.at[pl.ds(eb, CC)], sidx_v)
            pltpu.sync_copy(dst_hbm.at[pl.ds(eb, CC)], didx_v)
            pltpu.sync_copy(
                exh_hbm.at[pl.ds(2 * _hp * EPAD + eb, CC)],
                exh_v.at[0])
            pltpu.sync_copy(
                exh_hbm.at[pl.ds((2 * _hp + 1) * EPAD + eb, CC)],
                exh_v.at[1])

            for j in range(CC // 16):
                v = sidx_v[pl.ds(j * 16, 16)]
                gidx_v[pl.ds(j * 16, 16)] = v * (H // 2) + _hp

            pltpu.async_copy(fsh_hbm.at[gidx_v], rows_v, sem).wait()

            def scale_body(j, _):
                exa16 = exh_v[0, pl.ds(j * 16, 16)]
                exb16 = exh_v[1, pl.ds(j * 16, 16)]
                for e2 in range(16):
                    aa = _lane_bcast(exa16, e2)
                    ab = _lane_bcast(exb16, e2)
                    e = j * 16 + e2
                    for q in range(DH // 16):
                        rows_v[e, pl.ds(q * 16, 16)] = (
                            rows_v[e, pl.ds(q * 16, 16)] * aa)
                    for q in range(DH // 16, 2 * DH // 16):
                        rows_v[e, pl.ds(q * 16, 16)] = (
                            rows_v[e, pl.ds(q * 16, 16)] * ab)
                return 0

            lax.fori_loop(0, CC // 16, scale_body, 0)
            pltpu.sync_copy(rows_v, out_s.at[didx_v], add=True)
            return 0

        lax.fori_loop(0, EPW // CC, chunk_body, 0)
        plsc.subcore_barrier()
        pltpu.sync_copy(out_s.at[pl.ds(sid * RPT, RPT), :],
                        outp_hbm.at[pl.ds(cid * N + sid * RPT, RPT),
                                    pl.ds(hp * 2 * DH, 2 * DH)])


def _epi_body(final, o0_ref, o1_ref, d0_ref, d1_ref, hp_ref, bias_ref,
              g1_ref, b1_ref, mex_ref, g2_ref, b2_ref, out_ref):
    den = d0_ref[...] + d1_ref[...] + 1e-9
    dx = jnp.dot(den, mex_ref[...], preferred_element_type=jnp.float32)
    o = (o0_ref[...] + o1_ref[...]) / dx + bias_ref[...]
    o = jnp.where(o > 0, o, jnp.exp(o) - 1.0)
    t = o + hp_ref[...]
    mu = jnp.mean(t, axis=1, keepdims=True)
    var = jnp.mean((t - mu) * (t - mu), axis=1, keepdims=True)
    t = (t - mu) * lax.rsqrt(var + EPS) * g1_ref[...] + b1_ref[...]
    if final:
        mu2 = jnp.mean(t, axis=1, keepdims=True)
        var2 = jnp.mean((t - mu2) * (t - mu2), axis=1, keepdims=True)
        t = (t - mu2) * lax.rsqrt(var2 + EPS) * g2_ref[...] + b2_ref[...]
    out_ref[...] = t


def _epilogue(o0, o1, d0, d1, hprev, bias, g1, b1, mex, g2, b2, final):
    bm = 400
    row = pl.BlockSpec((bm, D), lambda i: (i, 0))
    vec = pl.BlockSpec((1, D), lambda i: (0, 0))
    return pl.pallas_call(
        functools.partial(_epi_body, final),
        grid=(N // bm,),
        in_specs=[
            row, row,
            pl.BlockSpec((bm, H), lambda i: (i, 0)),
            pl.BlockSpec((bm, H), lambda i: (i, 0)),
            row, vec, vec, vec,
            pl.BlockSpec((H, D), lambda i: (0, 0)),
            vec, vec,
        ],
        out_specs=row,
        out_shape=jax.ShapeDtypeStruct((N, D), jnp.float32),
    )(o0, o1, d0, d1, hprev, bias, g1, b1, mex, g2, b2)


def kernel(x, edge_index, W_src, W_dst, attn, gat_bias, ln_g, ln_b,
           outn_g, outn_b):
    src = edge_index[0].astype(jnp.int32)
    dst = edge_index[1].astype(jnp.int32)
    pad = jnp.zeros((EPAD - E,), jnp.int32)
    srcp = jnp.concatenate([src, pad])
    dstp = jnp.concatenate([dst, pad])
    zeros_nh = jnp.zeros((N, H), jnp.float32)
    zeros_nd = jnp.zeros((N, 2 * DH), jnp.float32)
    mex = jnp.repeat(jnp.eye(H, dtype=jnp.float32), DH, axis=1)

    h = x
    for l in range(L):
        fs, fd = _project(h, W_src[l], W_dst[l])
        scores = _score_kernel(fs, fd, srcp, dstp, attn[l].reshape(-1))
        exf = _softmax_prep(scores.reshape(-1)).reshape(-1)
        ex_em = _transpose_ex(exf.reshape(H, EPAD))
        denomp = _denom_kernel(ex_em, dstp, zeros_nh)
        outp = _message_kernel(fs.reshape(N * H // 2, 2 * DH), srcp, dstp,
                               exf, zeros_nd)
        h = _epilogue(outp[:N], outp[N:], denomp[:N], denomp[N:], h,
                      gat_bias[l].reshape(1, D), ln_g[l].reshape(1, D),
                      ln_b[l].reshape(1, D), mex,
                      outn_g.reshape(1, D), outn_b.reshape(1, D),
                      final=(l == L - 1))
    return h


# speedup vs baseline: 3.8695x; 1.2437x over previous
import functools

import jax
import jax.numpy as jnp
from jax import lax
from jax.experimental import pallas as pl
from jax.experimental.pallas import tpu as pltpu
from jax.experimental.pallas import tpu_sc as plsc

N = 10000
E = 160000
D = 512
H = 8
DH = 64
L = 2
EPS = 1e-5
NEG_SLOPE = 0.2

NC = 2
NS = 16
NW = NC * NS
EPAD = 163840
EPW = EPAD // NW
CA = 32
CB = 512
CC = 128
RPT = N // NS
RS = EPAD * H // 128

_sc_mesh = plsc.VectorSubcoreMesh(core_axis_name="c", subcore_axis_name="s")

_GDN = lax.GatherDimensionNumbers(
    offset_dims=(), collapsed_slice_dims=(0,), start_index_map=(0,))


def _lane_bcast(v16, i):
    idx = jnp.zeros((16,), jnp.int32) + i
    return lax.gather(v16, idx[:, None], _GDN, (1,),
                      mode=lax.GatherScatterMode.PROMISE_IN_BOUNDS)


def _lane_perm(v16, idx16):
    return lax.gather(v16, idx16[:, None], _GDN, (1,),
                      mode=lax.GatherScatterMode.PROMISE_IN_BOUNDS)


def _lane_sum(v16, perms):
    for pm in perms:
        v16 = v16 + _lane_perm(v16, pm)
    return v16


def _proj_body(x_ref, ws_ref, wd_ref, fs_ref, fd_ref):
    x = x_ref[...]
    fs_ref[...] = jnp.dot(x, ws_ref[...], preferred_element_type=jnp.float32)
    fd_ref[...] = jnp.dot(x, wd_ref[...], preferred_element_type=jnp.float32)


def _project(h, ws, wd):
    bm = 400
    return pl.pallas_call(
        _proj_body,
        grid=(N // bm,),
        in_specs=[
            pl.BlockSpec((bm, D), lambda i: (i, 0)),
            pl.BlockSpec((D, D), lambda i: (0, 0)),
            pl.BlockSpec((D, D), lambda i: (0, 0)),
        ],
        out_specs=[
            pl.BlockSpec((bm, D), lambda i: (i, 0)),
            pl.BlockSpec((bm, D), lambda i: (i, 0)),
        ],
        out_shape=[jax.ShapeDtypeStruct((N, D), jnp.float32)] * 2,
    )(h, ws, wd)


@functools.partial(
    pl.kernel,
    out_type=jax.ShapeDtypeStruct((H, EPAD), jnp.float32),
    mesh=_sc_mesh,
    scratch_types=[
        pltpu.VMEM((CA,), jnp.int32),
        pltpu.VMEM((CA,), jnp.int32),
        pltpu.VMEM((CA,), jnp.int32),
        pltpu.VMEM((CA,), jnp.int32),
        pltpu.VMEM((CA, D), jnp.float32),
        pltpu.VMEM((CA, D), jnp.float32),
        pltpu.VMEM((CA, D), jnp.float32),
        pltpu.VMEM((CA, D), jnp.float32),
        pltpu.VMEM((H * DH,), jnp.float32),
        pltpu.VMEM((H, CA), jnp.float32),
        pltpu.SemaphoreType.DMA,
        pltpu.SemaphoreType.DMA,
    ],
    compiler_params=pltpu.CompilerParams(use_tc_tiling_on_sc=False, needs_layout_passes=False),
)
def _score_kernel(fs_hbm, fd_hbm, src_hbm, dst_hbm, attn_hbm, scores_hbm,
                  sidx0, didx0, sidx1, didx1, fs0, fd0, fs1, fd1,
                  attn_v, sc_v, sem0, sem1):
    cid = lax.axis_index("c")
    sid = lax.axis_index("s")
    wid = cid * NS + sid
    ebase = wid * EPW
    NCH = EPW // CA
    pltpu.sync_copy(attn_hbm, attn_v)
    ilane = lax.iota(jnp.int32, 16)
    perms = [ilane ^ m for m in (1, 2, 4, 8)]
    sidxs = [sidx0, sidx1]
    didxs = [didx0, didx1]
    fss = [fs0, fs1]
    fds = [fd0, fd1]
    sems = [sem0, sem1]

    def issue(b, c):
        @pl.when(c < NCH)
        def _():
            eb = ebase + c * CA
            pltpu.sync_copy(src_hbm.at[pl.ds(eb, CA)], sidxs[b])
            pltpu.sync_copy(dst_hbm.at[pl.ds(eb, CA)], didxs[b])
            pltpu.async_copy(fs_hbm.at[sidxs[b]], fss[b], sems[b])
            pltpu.async_copy(fd_hbm.at[didxs[b]], fds[b], sems[b])

    def drain(b):
        pltpu.make_async_copy(fs_hbm.at[sidxs[b]], fss[b], sems[b]).wait()
        pltpu.make_async_copy(fd_hbm.at[didxs[b]], fds[b], sems[b]).wait()

    def compute(b, c):
        eb = ebase + c * CA
        fs_v = fss[b]
        fd_v = fds[b]

        def head_body(h, _):
            hbase = h * DH
            at = [attn_v[pl.ds(hbase + q * 16, 16)] for q in range(4)]

            def group_body(g, _):
                scores16 = jnp.zeros((16,), jnp.float32)
                for e2 in range(16):
                    e = g * 16 + e2
                    ps = []
                    for q in range(4):
                        a = fs_v[e, pl.ds(hbase + q * 16, 16)]
                        b_ = fd_v[e, pl.ds(hbase + q * 16, 16)]
                        z = a + b_
                        z = jnp.maximum(z, NEG_SLOPE * z)
                        ps.append(at[q] * z)
                    p = (ps[0] + ps[1]) + (ps[2] + ps[3])
                    p = _lane_sum(p, perms)
                    scores16 = jnp.where(ilane == e2, p, scores16)
                sc_v[h, pl.ds(g * 16, 16)] = scores16
                return 0

            lax.fori_loop(0, CA // 16, group_body, 0)
            return 0

        lax.fori_loop(0, H, head_body, 0)
        pltpu.sync_copy(sc_v, scores_hbm.at[:, pl.ds(eb, CA)])

    issue(0, 0)

    def pair_body(c2, _):
        c = c2 * 2
        issue(1, c + 1)
        drain(0)
        compute(0, c)
        issue(0, c + 2)
        drain(1)
        compute(1, c + 1)
        return 0

    lax.fori_loop(0, NCH // 2, pair_body, 0)


def _softmax_body(s_ref, ex_ref):
    HB = EPAD // 128
    EB = E // 128
    riot = lax.broadcasted_iota(jnp.int32, (HB, 128), 0)
    for h in range(H):
        s = s_ref[pl.ds(h * HB, HB), :]
        mh = jnp.max(s)
        exs = jnp.exp(s - mh)
        ex_ref[pl.ds(h * HB, HB), :] = jnp.where(riot < EB, exs, 0.0)


def _softmax_prep(scores):
    return pl.pallas_call(
        _softmax_body,
        out_shape=jax.ShapeDtypeStruct((RS, 128), jnp.float32),
    )(scores.reshape(RS, 128))


def _tr_body(x_ref, o_ref):
    o_ref[...] = x_ref[...].T


def _transpose_ex(ex_hm):
    bt = 8192
    return pl.pallas_call(
        _tr_body,
        grid=(EPAD // bt,),
        in_specs=[pl.BlockSpec((H, bt), lambda i: (0, i))],
        out_specs=pl.BlockSpec((bt, H), lambda i: (i, 0)),
        out_shape=jax.ShapeDtypeStruct((EPAD, H), jnp.float32),
    )(ex_hm)


@functools.partial(
    pl.kernel,
    out_type=jax.ShapeDtypeStruct((NC * N, H), jnp.float32),
    mesh=_sc_mesh,
    scratch_types=[
        pltpu.VMEM((CB,), jnp.int32),
        pltpu.VMEM((CB, H), jnp.float32),
        pltpu.VMEM_SHARED((N, H), jnp.float32),
    ],
    compiler_params=pltpu.CompilerParams(use_tc_tiling_on_sc=False, needs_layout_passes=False),
)
def _denom_kernel(ex_hbm, dst_hbm, zeros_hbm, denomp_hbm,
                  didx_v, ex_v, den_s):
    cid = lax.axis_index("c")
    sid = lax.axis_index("s")
    wid = cid * NS + sid
    pltpu.sync_copy(zeros_hbm.at[pl.ds(sid * RPT, RPT), :],
                    den_s.at[pl.ds(sid * RPT, RPT), :])
    plsc.subcore_barrier()

    def chunk_body(c, _):
        eb = wid * EPW + c * CB
        pltpu.sync_copy(dst_hbm.at[pl.ds(eb, CB)], didx_v)
        pltpu.sync_copy(ex_hbm.at[pl.ds(eb, CB), :], ex_v)
        pltpu.sync_copy(ex_v, den_s.at[didx_v], add=True)
        return 0

    lax.fori_loop(0, EPW // CB, chunk_body, 0)
    plsc.subcore_barrier()
    pltpu.sync_copy(den_s.at[pl.ds(sid * RPT, RPT), :],
                    denomp_hbm.at[pl.ds(cid * N + sid * RPT, RPT), :])


@functools.partial(
    pl.kernel,
    out_type=jax.ShapeDtypeStruct((NC * N, D), jnp.float32),
    mesh=_sc_mesh,
    scratch_types=[
        pltpu.VMEM((CC,), jnp.int32),
        pltpu.VMEM((CC,), jnp.int32),
        pltpu.VMEM((CC,), jnp.int32),
        pltpu.VMEM((2, CC), jnp.float32),
        pltpu.VMEM((CC, 2 * DH), jnp.float32),
        pltpu.VMEM_SHARED((N, 2 * DH), jnp.float32),
        pltpu.SemaphoreType.DMA,
    ],
    compiler_params=pltpu.CompilerParams(use_tc_tiling_on_sc=False, needs_layout_passes=False),
)
def _message_kernel(fsh_hbm, src_hbm, dst_hbm, exh_hbm, zeros_hbm, outp_hbm,
                    sidx_v, didx_v, gidx_v, exh_v, rows_v, out_s, sem):
    cid = lax.axis_index("c")
    sid = lax.axis_index("s")
    wid = cid * NS + sid
    for hp in range(H // 2):
        pltpu.sync_copy(zeros_hbm.at[pl.ds(sid * RPT, RPT), :],
                        out_s.at[pl.ds(sid * RPT, RPT), :])
        plsc.subcore_barrier()

        def chunk_body(c, _, _hp=hp):
            eb = wid * EPW + c * CC
            pltpu.sync_copy(src_hbm.at[pl.ds(eb, CC)], sidx_v)
            pltpu.sync_copy(dst_hbm.at[pl.ds(eb, CC)], didx_v)
            pltpu.sync_copy(
                exh_hbm.at[pl.ds(2 * _hp * EPAD + eb, CC)],
                exh_v.at[0])
            pltpu.sync_copy(
                exh_hbm.at[pl.ds((2 * _hp + 1) * EPAD + eb, CC)],
                exh_v.at[1])

            for j in range(CC // 16):
                v = sidx_v[pl.ds(j * 16, 16)]
                gidx_v[pl.ds(j * 16, 16)] = v * (H // 2) + _hp

            pltpu.async_copy(fsh_hbm.at[gidx_v], rows_v, sem).wait()

            def scale_body(j, _):
                exa16 = exh_v[0, pl.ds(j * 16, 16)]
                exb16 = exh_v[1, pl.ds(j * 16, 16)]
                for e2 in range(16):
                    aa = _lane_bcast(exa16, e2)
                    ab = _lane_bcast(exb16, e2)
                    e = j * 16 + e2
                    for q in range(DH // 16):
                        rows_v[e, pl.ds(q * 16, 16)] = (
                            rows_v[e, pl.ds(q * 16, 16)] * aa)
                    for q in range(DH // 16, 2 * DH // 16):
                        rows_v[e, pl.ds(q * 16, 16)] = (
                            rows_v[e, pl.ds(q * 16, 16)] * ab)
                return 0

            lax.fori_loop(0, CC // 16, scale_body, 0)
            pltpu.sync_copy(rows_v, out_s.at[didx_v], add=True)
            return 0

        lax.fori_loop(0, EPW // CC, chunk_body, 0)
        plsc.subcore_barrier()
        pltpu.sync_copy(out_s.at[pl.ds(sid * RPT, RPT), :],
                        outp_hbm.at[pl.ds(cid * N + sid * RPT, RPT),
                                    pl.ds(hp * 2 * DH, 2 * DH)])


def _epi_body(final, o0_ref, o1_ref, d0_ref, d1_ref, hp_ref, bias_ref,
              g1_ref, b1_ref, mex_ref, g2_ref, b2_ref, out_ref):
    den = d0_ref[...] + d1_ref[...] + 1e-9
    dx = jnp.dot(den, mex_ref[...], preferred_element_type=jnp.float32)
    o = (o0_ref[...] + o1_ref[...]) / dx + bias_ref[...]
    o = jnp.where(o > 0, o, jnp.exp(o) - 1.0)
    t = o + hp_ref[...]
    mu = jnp.mean(t, axis=1, keepdims=True)
    var = jnp.mean((t - mu) * (t - mu), axis=1, keepdims=True)
    t = (t - mu) * lax.rsqrt(var + EPS) * g1_ref[...] + b1_ref[...]
    if final:
        mu2 = jnp.mean(t, axis=1, keepdims=True)
        var2 = jnp.mean((t - mu2) * (t - mu2), axis=1, keepdims=True)
        t = (t - mu2) * lax.rsqrt(var2 + EPS) * g2_ref[...] + b2_ref[...]
    out_ref[...] = t


def _epilogue(o0, o1, d0, d1, hprev, bias, g1, b1, mex, g2, b2, final):
    bm = 400
    row = pl.BlockSpec((bm, D), lambda i: (i, 0))
    vec = pl.BlockSpec((1, D), lambda i: (0, 0))
    return pl.pallas_call(
        functools.partial(_epi_body, final),
        grid=(N // bm,),
        in_specs=[
            row, row,
            pl.BlockSpec((bm, H), lambda i: (i, 0)),
            pl.BlockSpec((bm, H), lambda i: (i, 0)),
            row, vec, vec, vec,
            pl.BlockSpec((H, D), lambda i: (0, 0)),
            vec, vec,
        ],
        out_specs=row,
        out_shape=jax.ShapeDtypeStruct((N, D), jnp.float32),
    )(o0, o1, d0, d1, hprev, bias, g1, b1, mex, g2, b2)


def kernel(x, edge_index, W_src, W_dst, attn, gat_bias, ln_g, ln_b,
           outn_g, outn_b):
    src = edge_index[0].astype(jnp.int32)
    dst = edge_index[1].astype(jnp.int32)
    pad = jnp.zeros((EPAD - E,), jnp.int32)
    srcp = jnp.concatenate([src, pad])
    dstp = jnp.concatenate([dst, pad])
    zeros_nh = jnp.zeros((N, H), jnp.float32)
    zeros_nd = jnp.zeros((N, 2 * DH), jnp.float32)
    mex = jnp.repeat(jnp.eye(H, dtype=jnp.float32), DH, axis=1)

    h = x
    for l in range(L):
        fs, fd = _project(h, W_src[l], W_dst[l])
        scores = _score_kernel(fs, fd, srcp, dstp, attn[l].reshape(-1))
        exf = _softmax_prep(scores.reshape(-1)).reshape(-1)
        ex_em = _transpose_ex(exf.reshape(H, EPAD))
        denomp = _denom_kernel(ex_em, dstp, zeros_nh)
        outp = _message_kernel(fs.reshape(N * H // 2, 2 * DH), srcp, dstp,
                               exf, zeros_nd)
        h = _epilogue(outp[:N], outp[N:], denomp[:N], denomp[N:], h,
                      gat_bias[l].reshape(1, D), ln_g[l].reshape(1, D),
                      ln_b[l].reshape(1, D), mex,
                      outn_g.reshape(1, D), outn_b.reshape(1, D),
                      final=(l == L - 1))
    return h


# speedup vs baseline: 4.7605x; 1.2303x over previous
import functools

import jax
import jax.numpy as jnp
from jax import lax
from jax.experimental import pallas as pl
from jax.experimental.pallas import tpu as pltpu
from jax.experimental.pallas import tpu_sc as plsc

N = 10000
E = 160000
D = 512
H = 8
DH = 64
L = 2
EPS = 1e-5
NEG_SLOPE = 0.2

NC = 2
NS = 16
NW = NC * NS
EPAD = 163840
EPW = EPAD // NW
CA = 32
CB = 512
CC = 128
RPT = N // NS
RS = EPAD * H // 128

_sc_mesh = plsc.VectorSubcoreMesh(core_axis_name="c", subcore_axis_name="s")

_GDN = lax.GatherDimensionNumbers(
    offset_dims=(), collapsed_slice_dims=(0,), start_index_map=(0,))


def _lane_bcast(v16, i):
    idx = jnp.zeros((16,), jnp.int32) + i
    return lax.gather(v16, idx[:, None], _GDN, (1,),
                      mode=lax.GatherScatterMode.PROMISE_IN_BOUNDS)


def _lane_perm(v16, idx16):
    return lax.gather(v16, idx16[:, None], _GDN, (1,),
                      mode=lax.GatherScatterMode.PROMISE_IN_BOUNDS)


def _lane_sum(v16, perms):
    for pm in perms:
        v16 = v16 + _lane_perm(v16, pm)
    return v16


def _proj_body(x_ref, ws_ref, wd_ref, fs_ref, fd_ref):
    x = x_ref[...]
    fs_ref[...] = jnp.dot(x, ws_ref[...], preferred_element_type=jnp.float32)
    fd_ref[...] = jnp.dot(x, wd_ref[...], preferred_element_type=jnp.float32)


def _project(h, ws, wd):
    bm = 400
    return pl.pallas_call(
        _proj_body,
        grid=(N // bm,),
        in_specs=[
            pl.BlockSpec((bm, D), lambda i: (i, 0)),
            pl.BlockSpec((D, D), lambda i: (0, 0)),
            pl.BlockSpec((D, D), lambda i: (0, 0)),
        ],
        out_specs=[
            pl.BlockSpec((bm, D), lambda i: (i, 0)),
            pl.BlockSpec((bm, D), lambda i: (i, 0)),
        ],
        out_shape=[jax.ShapeDtypeStruct((N, D), jnp.float32)] * 2,
    )(h, ws, wd)


@functools.partial(
    pl.kernel,
    out_type=jax.ShapeDtypeStruct((H, EPAD), jnp.float32),
    mesh=_sc_mesh,
    scratch_types=[
        pltpu.VMEM((CA,), jnp.int32),
        pltpu.VMEM((CA,), jnp.int32),
        pltpu.VMEM((CA,), jnp.int32),
        pltpu.VMEM((CA,), jnp.int32),
        pltpu.VMEM((CA, D), jnp.float32),
        pltpu.VMEM((CA, D), jnp.float32),
        pltpu.VMEM((CA, D), jnp.float32),
        pltpu.VMEM((CA, D), jnp.float32),
        pltpu.VMEM((H * DH,), jnp.float32),
        pltpu.VMEM((H, CA), jnp.float32),
        pltpu.SemaphoreType.DMA,
        pltpu.SemaphoreType.DMA,
    ],
    compiler_params=pltpu.CompilerParams(use_tc_tiling_on_sc=False, needs_layout_passes=False),
)
def _score_kernel(fs_hbm, fd_hbm, src_hbm, dst_hbm, attn_hbm, scores_hbm,
                  sidx0, didx0, sidx1, didx1, fs0, fd0, fs1, fd1,
                  attn_v, sc_v, sem0, sem1):
    cid = lax.axis_index("c")
    sid = lax.axis_index("s")
    wid = cid * NS + sid
    ebase = wid * EPW
    NCH = EPW // CA
    pltpu.sync_copy(attn_hbm, attn_v)
    ilane = lax.iota(jnp.int32, 16)
    perms = [ilane ^ m for m in (1, 2, 4, 8)]
    sidxs = [sidx0, sidx1]
    didxs = [didx0, didx1]
    fss = [fs0, fs1]
    fds = [fd0, fd1]
    sems = [sem0, sem1]

    def issue(b, c):
        @pl.when(c < NCH)
        def _():
            eb = ebase + c * CA
            pltpu.sync_copy(src_hbm.at[pl.ds(eb, CA)], sidxs[b])
            pltpu.sync_copy(dst_hbm.at[pl.ds(eb, CA)], didxs[b])
            pltpu.async_copy(fs_hbm.at[sidxs[b]], fss[b], sems[b])
            pltpu.async_copy(fd_hbm.at[didxs[b]], fds[b], sems[b])

    def drain(b):
        pltpu.make_async_copy(fs_hbm.at[sidxs[b]], fss[b], sems[b]).wait()
        pltpu.make_async_copy(fd_hbm.at[didxs[b]], fds[b], sems[b]).wait()

    def compute(b, c):
        eb = ebase + c * CA
        fs_v = fss[b]
        fd_v = fds[b]

        def head_body(h, _):
            hbase = h * DH
            at = [attn_v[pl.ds(hbase + q * 16, 16)] for q in range(4)]

            def group_body(g, _):
                scores16 = jnp.zeros((16,), jnp.float32)
                for e2 in range(16):
                    e = g * 16 + e2
                    ps = []
                    for q in range(4):
                        a = fs_v[e, pl.ds(hbase + q * 16, 16)]
                        b_ = fd_v[e, pl.ds(hbase + q * 16, 16)]
                        z = a + b_
                        z = jnp.maximum(z, NEG_SLOPE * z)
                        ps.append(at[q] * z)
                    p = (ps[0] + ps[1]) + (ps[2] + ps[3])
                    p = _lane_sum(p, perms)
                    scores16 = jnp.where(ilane == e2, p, scores16)
                sc_v[h, pl.ds(g * 16, 16)] = scores16
                return 0

            lax.fori_loop(0, CA // 16, group_body, 0)
            return 0

        lax.fori_loop(0, H, head_body, 0)
        pltpu.sync_copy(sc_v, scores_hbm.at[:, pl.ds(eb, CA)])

    issue(0, 0)

    def pair_body(c2, _):
        c = c2 * 2
        issue(1, c + 1)
        drain(0)
        compute(0, c)
        issue(0, c + 2)
        drain(1)
        compute(1, c + 1)
        return 0

    lax.fori_loop(0, NCH // 2, pair_body, 0)


def _softmax_body(s_ref, ex_ref):
    HB = EPAD // 128
    EB = E // 128
    riot = lax.broadcasted_iota(jnp.int32, (HB, 128), 0)
    for h in range(H):
        s = s_ref[pl.ds(h * HB, HB), :]
        mh = jnp.max(s)
        exs = jnp.exp(s - mh)
        ex_ref[pl.ds(h * HB, HB), :] = jnp.where(riot < EB, exs, 0.0)


def _softmax_prep(scores):
    return pl.pallas_call(
        _softmax_body,
        out_shape=jax.ShapeDtypeStruct((RS, 128), jnp.float32),
    )(scores.reshape(RS, 128))


def _tr_body(x_ref, o_ref):
    o_ref[...] = x_ref[...].T


def _transpose_ex(ex_hm):
    bt = 8192
    return pl.pallas_call(
        _tr_body,
        grid=(EPAD // bt,),
        in_specs=[pl.BlockSpec((H, bt), lambda i: (0, i))],
        out_specs=pl.BlockSpec((bt, H), lambda i: (i, 0)),
        out_shape=jax.ShapeDtypeStruct((EPAD, H), jnp.float32),
    )(ex_hm)


@functools.partial(
    pl.kernel,
    out_type=jax.ShapeDtypeStruct((NC * N, H), jnp.float32),
    mesh=_sc_mesh,
    scratch_types=[
        pltpu.VMEM((CB,), jnp.int32),
        pltpu.VMEM((CB, H), jnp.float32),
        pltpu.VMEM_SHARED((N, H), jnp.float32),
    ],
    compiler_params=pltpu.CompilerParams(use_tc_tiling_on_sc=False, needs_layout_passes=False),
)
def _denom_kernel(ex_hbm, dst_hbm, zeros_hbm, denomp_hbm,
                  didx_v, ex_v, den_s):
    cid = lax.axis_index("c")
    sid = lax.axis_index("s")
    wid = cid * NS + sid
    pltpu.sync_copy(zeros_hbm.at[pl.ds(sid * RPT, RPT), :],
                    den_s.at[pl.ds(sid * RPT, RPT), :])
    plsc.subcore_barrier()

    def chunk_body(c, _):
        eb = wid * EPW + c * CB
        pltpu.sync_copy(dst_hbm.at[pl.ds(eb, CB)], didx_v)
        pltpu.sync_copy(ex_hbm.at[pl.ds(eb, CB), :], ex_v)
        pltpu.sync_copy(ex_v, den_s.at[didx_v], add=True)
        return 0

    lax.fori_loop(0, EPW // CB, chunk_body, 0)
    plsc.subcore_barrier()
    pltpu.sync_copy(den_s.at[pl.ds(sid * RPT, RPT), :],
                    denomp_hbm.at[pl.ds(cid * N + sid * RPT, RPT), :])


@functools.partial(
    pl.kernel,
    out_type=jax.ShapeDtypeStruct((NC * N, D), jnp.float32),
    mesh=_sc_mesh,
    scratch_types=[
        pltpu.VMEM((2, CC), jnp.int32),
        pltpu.VMEM((2, CC), jnp.int32),
        pltpu.VMEM((2, CC), jnp.int32),
        pltpu.VMEM((2, 2, CC), jnp.float32),
        pltpu.VMEM((CC, 2 * DH), jnp.float32),
        pltpu.VMEM((CC, 2 * DH), jnp.float32),
        pltpu.VMEM_SHARED((N, 2 * DH), jnp.float32),
        pltpu.SemaphoreType.DMA,
        pltpu.SemaphoreType.DMA,
    ],
    compiler_params=pltpu.CompilerParams(use_tc_tiling_on_sc=False, needs_layout_passes=False),
)
def _message_kernel(fsh_hbm, src_hbm, dst_hbm, exh_hbm, zeros_hbm, outp_hbm,
                    sidx_v, didx_v, gidx_v, exh_v, rows0, rows1, out_s,
                    sem0, sem1):
    cid = lax.axis_index("c")
    sid = lax.axis_index("s")
    wid = cid * NS + sid
    NCH = EPW // CC
    rows = [rows0, rows1]
    sems = [sem0, sem1]
    for hp in range(H // 2):
        pltpu.sync_copy(zeros_hbm.at[pl.ds(sid * RPT, RPT), :],
                        out_s.at[pl.ds(sid * RPT, RPT), :])
        plsc.subcore_barrier()

        def issue(b, c, _hp=hp):
            @pl.when(c < NCH)
            def _():
                eb = wid * EPW + c * CC
                pltpu.sync_copy(src_hbm.at[pl.ds(eb, CC)],
                                sidx_v.at[b])
                pltpu.sync_copy(dst_hbm.at[pl.ds(eb, CC)],
                                didx_v.at[b])
                pltpu.sync_copy(
                    exh_hbm.at[pl.ds(2 * _hp * EPAD + eb, CC)],
                    exh_v.at[b, 0])
                pltpu.sync_copy(
                    exh_hbm.at[pl.ds((2 * _hp + 1) * EPAD + eb, CC)],
                    exh_v.at[b, 1])
                for j in range(CC // 16):
                    v = sidx_v[b, pl.ds(j * 16, 16)]
                    gidx_v[b, pl.ds(j * 16, 16)] = v * (H // 2) + _hp
                pltpu.async_copy(fsh_hbm.at[gidx_v.at[b]], rows[b],
                                 sems[b])

        def drain(b):
            pltpu.make_async_copy(fsh_hbm.at[gidx_v.at[b]], rows[b],
                                  sems[b]).wait()

        def compute(b):
            rows_v = rows[b]

            def scale_body(j, _):
                exa16 = exh_v[b, 0, pl.ds(j * 16, 16)]
                exb16 = exh_v[b, 1, pl.ds(j * 16, 16)]
                for e2 in range(16):
                    aa = _lane_bcast(exa16, e2)
                    ab = _lane_bcast(exb16, e2)
                    e = j * 16 + e2
                    for q in range(DH // 16):
                        rows_v[e, pl.ds(q * 16, 16)] = (
                            rows_v[e, pl.ds(q * 16, 16)] * aa)
                    for q in range(DH // 16, 2 * DH // 16):
                        rows_v[e, pl.ds(q * 16, 16)] = (
                            rows_v[e, pl.ds(q * 16, 16)] * ab)
                return 0

            lax.fori_loop(0, CC // 16, scale_body, 0)
            pltpu.sync_copy(rows_v, out_s.at[didx_v.at[b]], add=True)

        issue(0, 0)

        def pair_body(c2, _):
            c = c2 * 2
            issue(1, c + 1)
            drain(0)
            compute(0)
            issue(0, c + 2)
            drain(1)
            compute(1)
            return 0

        lax.fori_loop(0, NCH // 2, pair_body, 0)
        plsc.subcore_barrier()
        pltpu.sync_copy(out_s.at[pl.ds(sid * RPT, RPT), :],
                        outp_hbm.at[pl.ds(cid * N + sid * RPT, RPT),
                                    pl.ds(hp * 2 * DH, 2 * DH)])


def _epi_body(final, o0_ref, o1_ref, d0_ref, d1_ref, hp_ref, bias_ref,
              g1_ref, b1_ref, mex_ref, g2_ref, b2_ref, out_ref):
    den = d0_ref[...] + d1_ref[...] + 1e-9
    dx = jnp.dot(den, mex_ref[...], preferred_element_type=jnp.float32)
    o = (o0_ref[...] + o1_ref[...]) / dx + bias_ref[...]
    o = jnp.where(o > 0, o, jnp.exp(o) - 1.0)
    t = o + hp_ref[...]
    mu = jnp.mean(t, axis=1, keepdims=True)
    var = jnp.mean((t - mu) * (t - mu), axis=1, keepdims=True)
    t = (t - mu) * lax.rsqrt(var + EPS) * g1_ref[...] + b1_ref[...]
    if final:
        mu2 = jnp.mean(t, axis=1, keepdims=True)
        var2 = jnp.mean((t - mu2) * (t - mu2), axis=1, keepdims=True)
        t = (t - mu2) * lax.rsqrt(var2 + EPS) * g2_ref[...] + b2_ref[...]
    out_ref[...] = t


def _epilogue(o0, o1, d0, d1, hprev, bias, g1, b1, mex, g2, b2, final):
    bm = 400
    row = pl.BlockSpec((bm, D), lambda i: (i, 0))
    vec = pl.BlockSpec((1, D), lambda i: (0, 0))
    return pl.pallas_call(
        functools.partial(_epi_body, final),
        grid=(N // bm,),
        in_specs=[
            row, row,
            pl.BlockSpec((bm, H), lambda i: (i, 0)),
            pl.BlockSpec((bm, H), lambda i: (i, 0)),
            row, vec, vec, vec,
            pl.BlockSpec((H, D), lambda i: (0, 0)),
            vec, vec,
        ],
        out_specs=row,
        out_shape=jax.ShapeDtypeStruct((N, D), jnp.float32),
    )(o0, o1, d0, d1, hprev, bias, g1, b1, mex, g2, b2)


def kernel(x, edge_index, W_src, W_dst, attn, gat_bias, ln_g, ln_b,
           outn_g, outn_b):
    src = edge_index[0].astype(jnp.int32)
    dst = edge_index[1].astype(jnp.int32)
    pad = jnp.zeros((EPAD - E,), jnp.int32)
    srcp = jnp.concatenate([src, pad])
    dstp = jnp.concatenate([dst, pad])
    zeros_nh = jnp.zeros((N, H), jnp.float32)
    zeros_nd = jnp.zeros((N, 2 * DH), jnp.float32)
    mex = jnp.repeat(jnp.eye(H, dtype=jnp.float32), DH, axis=1)

    h = x
    for l in range(L):
        fs, fd = _project(h, W_src[l], W_dst[l])
        scores = _score_kernel(fs, fd, srcp, dstp, attn[l].reshape(-1))
        exf = _softmax_prep(scores.reshape(-1)).reshape(-1)
        ex_em = _transpose_ex(exf.reshape(H, EPAD))
        denomp = _denom_kernel(ex_em, dstp, zeros_nh)
        outp = _message_kernel(fs.reshape(N * H // 2, 2 * DH), srcp, dstp,
                               exf, zeros_nd)
        h = _epilogue(outp[:N], outp[N:], denomp[:N], denomp[N:], h,
                      gat_bias[l].reshape(1, D), ln_g[l].reshape(1, D),
                      ln_b[l].reshape(1, D), mex,
                      outn_g.reshape(1, D), outn_b.reshape(1, D),
                      final=(l == L - 1))
    return h
